# Initial kernel scaffold; baseline (speedup 1.0000x reference)
import jax
import jax.numpy as jnp
from jax.experimental import pallas as pl


def kernel(energy, target):
    raise NotImplementedError("write your pallas kernel here")


# speedup vs baseline: 2.4673x; 2.4673x over previous
import jax
import jax.numpy as jnp
from jax.experimental import pallas as pl
from jax.experimental.pallas import tpu as pltpu

N_NEG = 10
K = 30
BIGI = 2**30


def _sampler_kernel(t_ref, x_ref, g_ref, oi_ref, oe_ref):
    b = pl.program_id(0)
    R, C = x_ref.shape[1], x_ref.shape[2]
    ciota = jax.lax.broadcasted_iota(jnp.int32, (1, C), 1)
    riota = jax.lax.broadcasted_iota(jnp.int32, (1, R), 1)
    kiota = jax.lax.broadcasted_iota(jnp.int32, (1, K), 1)

    t = t_ref[b]
    rt = t // C
    ct = t - rt * C
    row_t = x_ref[0, pl.ds(rt, 1), :]
    x_ref[0, pl.ds(rt, 1), :] = jnp.where(ciota == ct, jnp.inf, row_t)

    m = jnp.min(x_ref[0], axis=1)[None, :]

    def body(j, carry):
        m, ti, te = carry
        v = jnp.min(m)
        r = jnp.min(jnp.where(m == v, riota, BIGI))
        row = x_ref[0, pl.ds(r, 1), :]
        c = jnp.min(jnp.where(row == v, ciota, BIGI))
        ti = jnp.where(kiota == j, r * C + c, ti)
        te = jnp.where(kiota == j, v, te)
        new_row = jnp.where(ciota == c, jnp.inf, row)
        x_ref[0, pl.ds(r, 1), :] = new_row
        m = jnp.where(riota == r, jnp.min(new_row), m)
        return m, ti, te

    ti0 = jnp.zeros((1, K), jnp.int32)
    te0 = jnp.zeros((1, K), jnp.float32)
    _, ti, te = jax.lax.fori_loop(0, K, body, (m, ti0, te0))

    g = g_ref[0]
    z = g - te
    zmax = jnp.max(z, axis=1, keepdims=True)
    s = jnp.min(jnp.where(z == zmax, kiota, BIGI), axis=1, keepdims=True)
    sel = kiota == s
    oi_ref[0, 0, :] = jnp.sum(jnp.where(sel, ti, 0), axis=1)
    oe_ref[0, 0, :] = jnp.sum(jnp.where(sel, te, 0.0), axis=1)


def kernel(energy, target):
    B, V = energy.shape
    C = 1000
    R = V // C
    x = energy.reshape(B, R, C)

    keys = jax.random.split(jax.random.key(1), B)
    gumbel = jax.vmap(lambda k: jax.random.gumbel(k, (N_NEG, K), jnp.float32))(keys)

    grid_spec = pltpu.PrefetchScalarGridSpec(
        num_scalar_prefetch=1,
        grid=(B,),
        in_specs=[
            pl.BlockSpec((1, R, C), lambda b, t: (b, 0, 0)),
            pl.BlockSpec((1, N_NEG, K), lambda b, t: (b, 0, 0)),
        ],
        out_specs=[
            pl.BlockSpec((1, 1, N_NEG), lambda b, t: (b, 0, 0)),
            pl.BlockSpec((1, 1, N_NEG), lambda b, t: (b, 0, 0)),
        ],
    )
    oi, oe = pl.pallas_call(
        _sampler_kernel,
        grid_spec=grid_spec,
        out_shape=[
            jax.ShapeDtypeStruct((B, 1, N_NEG), jnp.int32),
            jax.ShapeDtypeStruct((B, 1, N_NEG), jnp.float32),
        ],
    )(target.astype(jnp.int32), x, gumbel)
    return (oi, oe)


# speedup vs baseline: 5.0537x; 2.0483x over previous
import jax
import jax.numpy as jnp
from jax.experimental import pallas as pl
from jax.experimental.pallas import tpu as pltpu

N_NEG = 10
K = 30
NSEL = 32
BIGI = 2**30
R = 1000
C = 1000


def _rowmin_kernel(x_ref, m_ref):
    m_ref[0, 0, :] = jnp.min(x_ref[0], axis=1)


def _selrows_kernel(m_ref, rl_ref):
    mm = m_ref[:, 0, :]
    riota = jax.lax.broadcasted_iota(jnp.int32, mm.shape, 1)
    jiota = jax.lax.broadcasted_iota(jnp.int32, (1, NSEL), 1)

    def body(j, carry):
        mm, rl = carry
        v = jnp.min(mm, axis=1, keepdims=True)
        r = jnp.min(jnp.where(mm == v, riota, BIGI), axis=1, keepdims=True)
        rl = jnp.where(jiota == j, r, rl)
        mm = jnp.where(riota == r, jnp.inf, mm)
        return mm, rl

    B = mm.shape[0]
    rl0 = jnp.zeros((B, NSEL), jnp.int32)
    _, rl = jax.lax.fori_loop(0, NSEL, body, (mm, rl0))
    rl_ref[...] = rl


def _gather_select_kernel(rl_smem, x_any, rlv_ref, ct_ref, tmask_ref, g_ref,
                          oi_ref, oe_ref, cand_ref, sem):
    B = rlv_ref.shape[0]

    def dma_start(t, _):
        b = t // NSEL
        j = t - b * NSEL
        rr = rl_smem[b, j]
        pltpu.make_async_copy(x_any.at[b, rr], cand_ref.at[b, j], sem).start()
        return 0

    jax.lax.fori_loop(0, B * NSEL, dma_start, 0)

    def dma_wait(t, _):
        b = t // NSEL
        j = t - b * NSEL
        rr = rl_smem[b, j]
        pltpu.make_async_copy(x_any.at[b, rr], cand_ref.at[b, j], sem).wait()
        return 0

    jax.lax.fori_loop(0, B * NSEL, dma_wait, 0)

    ciota3 = jax.lax.broadcasted_iota(jnp.int32, (B, NSEL, C), 2)
    rlv = rlv_ref[...]
    cond3 = (tmask_ref[...][:, :, None] == 1) & (ciota3 == ct_ref[...][:, None])
    cand_ref[...] = jnp.where(cond3, jnp.inf, cand_ref[...])

    cl = cand_ref[...]
    m1 = jnp.min(cl, axis=2)
    c1 = jnp.min(jnp.where(cl == m1[:, :, None], ciota3, BIGI), axis=2)
    t2 = jnp.where(ciota3 == c1[:, :, None], jnp.inf, cl)
    m2 = jnp.min(t2, axis=2)
    c2 = jnp.min(jnp.where(t2 == m2[:, :, None], ciota3, BIGI), axis=2)

    jiota = jax.lax.broadcasted_iota(jnp.int32, (B, NSEL), 1)
    k30 = jax.lax.broadcasted_iota(jnp.int32, (1, K), 1)

    def clean(args):
        m1, c1, m2, c2, ti = args
        g3 = rlv[:, :, None] * C + ciota3
        ex = jnp.zeros((B, NSEL, C), jnp.bool_)
        for s in range(K):
            ex = ex | (g3 == ti[:, s][:, None, None])
        cl = jnp.where(ex, jnp.inf, cand_ref[...])
        nm1 = jnp.min(cl, axis=2)
        nc1 = jnp.min(jnp.where(cl == nm1[:, :, None], ciota3, BIGI), axis=2)
        t2 = jnp.where(ciota3 == nc1[:, :, None], jnp.inf, cl)
        nm2 = jnp.min(t2, axis=2)
        nc2 = jnp.min(jnp.where(t2 == nm2[:, :, None], ciota3, BIGI), axis=2)
        return nm1, nc1, nm2, nc2, ti

    def body(k, carry):
        m1, c1, m2, c2, ti, te = carry
        stale = jnp.min(m1) == -jnp.inf
        m1, c1, m2, c2, ti = jax.lax.cond(stale, clean, lambda a: a,
                                          (m1, c1, m2, c2, ti))
        v = jnp.min(m1, axis=1, keepdims=True)
        rbest = jnp.min(jnp.where(m1 == v, rlv, BIGI), axis=1, keepdims=True)
        jstar = jnp.min(jnp.where((m1 == v) & (rlv == rbest), jiota, BIGI),
                        axis=1, keepdims=True)
        cstar = jnp.min(jnp.where(jiota == jstar, c1, BIGI), axis=1,
                        keepdims=True)
        ti = jnp.where(k30 == k, rbest * C + cstar, ti)
        te = jnp.where(k30 == k, v, te)
        sel = jiota == jstar
        promo = jnp.where(m2 == jnp.inf, -jnp.inf, m2)
        m1 = jnp.where(sel, promo, m1)
        c1 = jnp.where(sel, c2, c1)
        m2 = jnp.where(sel, jnp.inf, m2)
        c2 = jnp.where(sel, BIGI, c2)
        return m1, c1, m2, c2, ti, te

    ti0 = jnp.full((B, K), BIGI, jnp.int32)
    te0 = jnp.zeros((B, K), jnp.float32)
    _, _, _, _, ti, te = jax.lax.fori_loop(0, K, body,
                                           (m1, c1, m2, c2, ti0, te0))

    g = g_ref[...]
    z = g - te[:, None, :]
    zmax = jnp.max(z, axis=2, keepdims=True)
    k30_3 = jax.lax.broadcasted_iota(jnp.int32, (B, N_NEG, K), 2)
    s = jnp.min(jnp.where(z == zmax, k30_3, BIGI), axis=2, keepdims=True)
    sel = k30_3 == s
    oi_ref[:, 0, :] = jnp.sum(jnp.where(sel, ti[:, None, :], 0), axis=2)
    oe_ref[:, 0, :] = jnp.sum(jnp.where(sel, te[:, None, :], 0.0), axis=2)


def kernel(energy, target):
    B, V = energy.shape
    x = energy.reshape(B, R, C)
    t32 = target.astype(jnp.int32)
    rt = t32 // C
    ct = t32 - rt * C

    keys = jax.random.split(jax.random.key(1), B)
    gumbel = jax.vmap(lambda k: jax.random.gumbel(k, (N_NEG, K), jnp.float32))(keys)

    m = pl.pallas_call(
        _rowmin_kernel,
        grid=(B,),
        in_specs=[pl.BlockSpec((1, R, C), lambda b: (b, 0, 0))],
        out_specs=pl.BlockSpec((1, 1, R), lambda b: (b, 0, 0)),
        out_shape=jax.ShapeDtypeStruct((B, 1, R), jnp.float32),
    )(x)

    rl = pl.pallas_call(
        _selrows_kernel,
        in_specs=[pl.BlockSpec((B, 1, R), lambda: (0, 0, 0))],
        out_specs=pl.BlockSpec((B, NSEL), lambda: (0, 0)),
        out_shape=jax.ShapeDtypeStruct((B, NSEL), jnp.int32),
    )(m)

    tmask = (rl == rt[:, None]).astype(jnp.int32)

    grid_spec = pltpu.PrefetchScalarGridSpec(
        num_scalar_prefetch=1,
        grid=(1,),
        in_specs=[
            pl.BlockSpec(memory_space=pl.ANY),
            pl.BlockSpec((B, NSEL), lambda i, rl_s: (0, 0)),
            pl.BlockSpec((B, 1), lambda i, rl_s: (0, 0)),
            pl.BlockSpec((B, NSEL), lambda i, rl_s: (0, 0)),
            pl.BlockSpec((B, N_NEG, K), lambda i, rl_s: (0, 0, 0)),
        ],
        out_specs=[
            pl.BlockSpec((B, 1, N_NEG), lambda i, rl_s: (0, 0, 0)),
            pl.BlockSpec((B, 1, N_NEG), lambda i, rl_s: (0, 0, 0)),
        ],
        scratch_shapes=[
            pltpu.VMEM((B, NSEL, C), jnp.float32),
            pltpu.SemaphoreType.DMA,
        ],
    )
    oi, oe = pl.pallas_call(
        _gather_select_kernel,
        grid_spec=grid_spec,
        out_shape=[
            jax.ShapeDtypeStruct((B, 1, N_NEG), jnp.int32),
            jax.ShapeDtypeStruct((B, 1, N_NEG), jnp.float32),
        ],
    )(rl, x, rl, ct[:, None], tmask, gumbel)
    return (oi, oe)
